# Initial kernel scaffold; baseline (speedup 1.0000x reference)
import jax
import jax.numpy as jnp
from jax.experimental import pallas as pl


def kernel(idx, W1, W2, W3):
    raise NotImplementedError("write your pallas kernel here")


# speedup vs baseline: 1.8667x; 1.8667x over previous
import functools

import jax
import jax.numpy as jnp
from jax import lax
from jax.experimental import pallas as pl
from jax.experimental.pallas import tpu as pltpu
from jax.experimental.pallas import tpu_sc as plsc

BATCH, HIST = 4096, 50
N = BATCH * HIST
D = 64
DOUT = 3 * D
CHUNK = 128
L = 16

_INFO = plsc.get_sparse_core_info()
NC, NS = _INFO.num_cores, _INFO.num_subcores
NW = NC * NS
PER_W = N // NW
STEPS = PER_W // CHUNK
GRP = CHUNK // L

_R1, _R2 = 366, 24


def _c(v):
    return jnp.full((L,), v, jnp.int32)


def _scale_from_normsq(nsq):
    i = lax.bitcast_convert_type(nsq, jnp.int32)
    i = 0x5F3759DF - lax.shift_right_logical(i, 1)
    r = lax.bitcast_convert_type(i, jnp.float32)
    half = nsq * 0.5
    r = r * (1.5 - half * r * r)
    r = r * (1.5 - half * r * r)
    r = r * (1.5 - half * r * r)
    norm = nsq * r
    return jnp.where(norm > 1.0, 1.0 / (norm + 1e-7), jnp.float32(1.0))


def _precompute_scales(tblf_v, nrows, scl_v):
    ngrp = (nrows + L - 1) // L

    def g_body(g, _):
        ridx = jnp.minimum(lax.iota(jnp.int32, L) + g * L, nrows - 1) * D

        def d_body(d, acc):
            v = plsc.load_gather(tblf_v, [ridx + d])
            return acc + v * v

        nsq = lax.fori_loop(0, D, d_body, jnp.zeros((L,), jnp.float32))
        scl_v[pl.ds(g * L, L)] = _scale_from_normsq(nsq)
        return 0

    lax.fori_loop(0, ngrp, g_body, 0)


def _body(idx_hbm, w1_hbm, w2_hbm, w3_hbm, out_hbm,
          w1_v, w2_v, scl1_v, scl2_v, idxf_v, i3_v, rows_v, out_v, sem):
    wid = lax.axis_index("s") * NC + lax.axis_index("c")

    pltpu.sync_copy(w1_hbm, w1_v)
    pltpu.sync_copy(w2_hbm, w2_v)
    _precompute_scales(w1_v, _R1, scl1_v)
    _precompute_scales(w2_v, _R2, scl2_v)

    def step(s, _):
        base = wid * PER_W + s * CHUNK
        pltpu.sync_copy(idx_hbm.at[pl.ds(base, CHUNK)], idxf_v)

        def i3_body(j, _):
            x = idxf_v[pl.ds(j * L, L)]
            i3_v[pl.ds(j * L, L)] = (x / 10.0).astype(jnp.int32)
            return 0

        lax.fori_loop(0, GRP, i3_body, 0)
        pltpu.async_copy(w3_hbm.at[i3_v], rows_v, sem).wait()

        def j_body(j, _):
            obase = j * L * DOUT + lax.iota(jnp.int32, L) * DOUT
            x = idxf_v[pl.ds(j * L, L)]
            idx1 = (x * 24.0).astype(jnp.int32)
            i1 = lax.rem(lax.div(idx1, 24), _R1)
            i2 = lax.rem(idx1, _R2)
            s1 = plsc.load_gather(scl1_v, [i1])
            s2 = plsc.load_gather(scl2_v, [i2])
            f1 = i1 * D
            f2 = i2 * D
            for d in range(D):
                v1 = plsc.load_gather(w1_v, [f1 + d])
                plsc.store_scatter(out_v, [obase + d], v1 * s1)
                v2 = plsc.load_gather(w2_v, [f2 + d])
                plsc.store_scatter(out_v, [obase + (D + d)], v2 * s2)
            return 0

        lax.fori_loop(0, GRP, j_body, 0)

        def r_body(r, _):
            a = [rows_v[r, pl.ds(dd * L, L)] for dd in range(D // L)]
            nsq = jnp.sum(a[0] * a[0] + a[1] * a[1] + a[2] * a[2] + a[3] * a[3])
            s3 = _scale_from_normsq(jnp.full((L,), nsq))
            ob = r * DOUT + 2 * D
            for dd in range(D // L):
                out_v[pl.ds(ob + dd * L, L)] = a[dd] * s3
            return 0

        lax.fori_loop(0, CHUNK, r_body, 0)
        pltpu.sync_copy(out_v, out_hbm.at[pl.ds(base * DOUT, CHUNK * DOUT)])
        return 0

    lax.fori_loop(0, STEPS, step, 0)


_sc_embed = functools.partial(
    pl.kernel,
    out_type=jax.ShapeDtypeStruct((N * DOUT,), jnp.float32),
    mesh=plsc.VectorSubcoreMesh(core_axis_name="c", subcore_axis_name="s"),
    compiler_params=pltpu.CompilerParams(
        needs_layout_passes=False, use_tc_tiling_on_sc=False),
    scratch_types=[
        pltpu.VMEM((_R1 * D,), jnp.float32),
        pltpu.VMEM((_R2 * D,), jnp.float32),
        pltpu.VMEM((384,), jnp.float32),
        pltpu.VMEM((32,), jnp.float32),
        pltpu.VMEM((CHUNK,), jnp.float32),
        pltpu.VMEM((CHUNK,), jnp.int32),
        pltpu.VMEM((CHUNK, D), jnp.float32),
        pltpu.VMEM((CHUNK * DOUT,), jnp.float32),
        pltpu.SemaphoreType.DMA,
    ],
)(_body)


def kernel(idx, W1, W2, W3):
    out = _sc_embed(idx.reshape(N), W1.reshape(_R1 * D), W2.reshape(_R2 * D), W3)
    return out.reshape(BATCH, HIST, DOUT)


# speedup vs baseline: 2.4374x; 1.3057x over previous
import functools

import jax
import jax.numpy as jnp
from jax import lax
from jax.experimental import pallas as pl
from jax.experimental.pallas import tpu as pltpu
from jax.experimental.pallas import tpu_sc as plsc

BATCH, HIST = 4096, 50
N = BATCH * HIST
D = 64
DOUT = 3 * D
CHUNK = 128
L = 16

_INFO = plsc.get_sparse_core_info()
NC, NS = _INFO.num_cores, _INFO.num_subcores
NW = NC * NS
NBT = BATCH // CHUNK
NCHUNK = HIST * NBT
STEPS = NCHUNK // NW
GRP = CHUNK // L
FT = DOUT // 8

_R1, _R2 = 366, 24


def _cc(v):
    return jnp.full((L,), v, jnp.int32)


def _scale_from_normsq(nsq):
    i = lax.bitcast_convert_type(nsq, jnp.int32)
    i = 0x5F3759DF - lax.shift_right_logical(i, 1)
    r = lax.bitcast_convert_type(i, jnp.float32)
    half = nsq * 0.5
    r = r * (1.5 - half * r * r)
    r = r * (1.5 - half * r * r)
    r = r * (1.5 - half * r * r)
    norm = nsq * r
    return jnp.where(norm > 1.0, 1.0 / (norm + 1e-7), jnp.float32(1.0))


def _prescale_rows(tblf_v, nrows):
    ngrp = (nrows + L - 1) // L

    def g_body(g, _):
        base = jnp.minimum(lax.iota(jnp.int32, L) + g * L, nrows - 1) * D

        def d_body(d, acc):
            v = plsc.load_gather(tblf_v, [base + d])
            return acc + v * v

        nsq = lax.fori_loop(0, D, d_body, jnp.zeros((L,), jnp.float32))
        s = _scale_from_normsq(nsq)

        def d_body2(d, _):
            v = plsc.load_gather(tblf_v, [base + d])
            plsc.store_scatter(tblf_v, [base + d], v * s)
            return 0

        lax.fori_loop(0, D, d_body2, 0)
        return 0

    lax.fori_loop(0, ngrp, g_body, 0)


def _body(idx_hbm, w1_hbm, w2_hbm, w3_hbm, out_hbm,
          w1_v, w2_v, idxf_v, i3_v, rows_v, out_v, sem):
    wid = lax.axis_index("s") * NC + lax.axis_index("c")

    pltpu.sync_copy(w1_hbm, w1_v)
    pltpu.sync_copy(w2_hbm, w2_v)
    _prescale_rows(w1_v, _R1)
    _prescale_rows(w2_v, _R2)

    def step(s, _):
        c = wid * STEPS + s
        h = c // NBT
        bt = c - h * NBT
        pltpu.sync_copy(idx_hbm.at[pl.ds(h * BATCH + bt * CHUNK, CHUNK)],
                        idxf_v)

        def i3_body(j, _):
            x = idxf_v[pl.ds(j * L, L)]
            i3_v[pl.ds(j * L, L)] = (x / 10.0).astype(jnp.int32)
            return 0

        lax.fori_loop(0, GRP, i3_body, 0)
        pltpu.async_copy(w3_hbm.at[i3_v], rows_v, sem).wait()

        def j_body(j, _):
            x = idxf_v[pl.ds(j * L, L)]
            idx1 = (x * 24.0).astype(jnp.int32)
            i1 = lax.rem(lax.div(idx1, 24), _R1) * D
            i2 = lax.rem(idx1, _R2) * D
            lanes = lax.iota(jnp.int32, L) + j * L
            ob = j * L
            for d in range(D):
                v1 = plsc.load_gather(w1_v, [i1 + d])
                out_v[d // 8, d % 8, pl.ds(ob, L)] = v1
                v2 = plsc.load_gather(w2_v, [i2 + d])
                f = D + d
                out_v[f // 8, f % 8, pl.ds(ob, L)] = v2
            acc = jnp.zeros((L,), jnp.float32)
            for d in range(D):
                v = plsc.load_gather(rows_v, [lanes, _cc(d)])
                acc = acc + v * v
            s3 = _scale_from_normsq(acc)
            for d in range(D):
                v3 = plsc.load_gather(rows_v, [lanes, _cc(d)])
                f = 2 * D + d
                out_v[f // 8, f % 8, pl.ds(ob, L)] = v3 * s3
            return 0

        lax.fori_loop(0, GRP, j_body, 0)
        pltpu.sync_copy(out_v, out_hbm.at[h, :, bt])
        return 0

    lax.fori_loop(0, STEPS, step, 0)


_sc_embed = functools.partial(
    pl.kernel,
    out_type=jax.ShapeDtypeStruct((HIST, FT, NBT, 8, CHUNK), jnp.float32),
    mesh=plsc.VectorSubcoreMesh(core_axis_name="c", subcore_axis_name="s"),
    compiler_params=pltpu.CompilerParams(
        needs_layout_passes=False, use_tc_tiling_on_sc=False),
    scratch_types=[
        pltpu.VMEM((_R1 * D,), jnp.float32),
        pltpu.VMEM((_R2 * D,), jnp.float32),
        pltpu.VMEM((CHUNK,), jnp.float32),
        pltpu.VMEM((CHUNK,), jnp.int32),
        pltpu.VMEM((CHUNK, D), jnp.float32),
        pltpu.VMEM((FT, 8, CHUNK), jnp.float32),
        pltpu.SemaphoreType.DMA,
    ],
)(_body)


def kernel(idx, W1, W2, W3):
    idx, W1, W2, W3 = lax.optimization_barrier((idx, W1, W2, W3))
    idxt = idx.reshape(BATCH, HIST).T.reshape(N)
    out5 = _sc_embed(idxt, W1.reshape(_R1 * D), W2.reshape(_R2 * D), W3)
    return out5.transpose(2, 4, 0, 1, 3).reshape(BATCH, HIST, DOUT)


# speedup vs baseline: 34.9868x; 14.3543x over previous
import functools

import jax
import jax.numpy as jnp
from jax import lax
from jax.experimental import pallas as pl
from jax.experimental.pallas import tpu as pltpu
from jax.experimental.pallas import tpu_sc as plsc

BATCH, HIST = 4096, 50
N = BATCH * HIST
D = 64
DOUT = 3 * D
CHUNK = 128
L = 16

_INFO = plsc.get_sparse_core_info()
NC, NS = _INFO.num_cores, _INFO.num_subcores
NW = NC * NS
NBT = BATCH // CHUNK
NCHUNK = HIST * NBT
STEPS = NCHUNK // NW
PER_W = N // NW
GRP = CHUNK // L
FT = DOUT // 8

_R1, _R2 = 366, 24


def _cc(v):
    return jnp.full((L,), v, jnp.int32)


def _divmod_c(n, c):
    q = (n.astype(jnp.float32) * jnp.float32(1.0 / c)).astype(jnp.int32)
    q = jnp.where(q * c > n, q - 1, q)
    q = jnp.where(q * c + c <= n, q + 1, q)
    return q, n - q * c


def _scale_from_normsq(nsq):
    i = lax.bitcast_convert_type(nsq, jnp.int32)
    i = 0x5F3759DF - lax.shift_right_logical(i, 1)
    r = lax.bitcast_convert_type(i, jnp.float32)
    half = nsq * 0.5
    r = r * (1.5 - half * r * r)
    r = r * (1.5 - half * r * r)
    r = r * (1.5 - half * r * r)
    norm = nsq * r
    return jnp.where(norm > 1.0, 1.0 / (norm + 1e-7), jnp.float32(1.0))


def _prescale_rows(tblf_v, nrows):
    ngrp = (nrows + L - 1) // L

    def g_body(g, _):
        base = jnp.minimum(lax.iota(jnp.int32, L) + g * L, nrows - 1) * D

        def d_body(d, acc):
            v = plsc.load_gather(tblf_v, [base + d])
            return acc + v * v

        nsq = lax.fori_loop(0, D, d_body, jnp.zeros((L,), jnp.float32))
        s = _scale_from_normsq(nsq)

        def d_body2(d, _):
            v = plsc.load_gather(tblf_v, [base + d])
            plsc.store_scatter(tblf_v, [base + d], v * s)
            return 0

        lax.fori_loop(0, D, d_body2, 0)
        return 0

    lax.fori_loop(0, ngrp, g_body, 0)


def _body(idx_hbm, w1_hbm, w2_hbm, w3_hbm, out_hbm,
          w1_v, w2_v, idx_v, i3_v, rows0_v, rows1_v, out0_v, out1_v,
          gsem0, gsem1, osem0, osem1):
    wid = lax.axis_index("s") * NC + lax.axis_index("c")
    rows_b = (rows0_v, rows1_v)
    out_b = (out0_v, out1_v)
    gsem_b = (gsem0, gsem1)
    osem_b = (osem0, osem1)

    pltpu.sync_copy(w1_hbm, w1_v)
    pltpu.sync_copy(w2_hbm, w2_v)
    pltpu.sync_copy(idx_hbm.at[pl.ds(wid * PER_W, PER_W)], idx_v)
    _prescale_rows(w1_v, _R1)
    _prescale_rows(w2_v, _R2)

    def i3_outer(s, _):
        def inner(j, _):
            x = idx_v[pl.ds(s * CHUNK + j * L, L)]
            i3_v[s, pl.ds(j * L, L)] = (x / 10.0).astype(jnp.int32)
            return 0
        lax.fori_loop(0, GRP, inner, 0)
        return 0

    lax.fori_loop(0, STEPS, i3_outer, 0)

    def _dst(s):
        c = wid * STEPS + s
        h = c // NBT
        bt = c - h * NBT
        return out_hbm.at[h, :, bt]

    def _gather(s, b):
        return pltpu.make_async_copy(
            w3_hbm.at[i3_v.at[s]], rows_b[b], gsem_b[b])

    _gather(0, 0).start()

    def compute(s, b):
        rows_v = rows_b[b]
        out_v = out_b[b]

        def j_body(j, _):
            x = idx_v[pl.ds(s * CHUNK + j * L, L)]
            idx1 = (x * 24.0).astype(jnp.int32)
            q, r2 = _divmod_c(idx1, 24)
            i1 = _divmod_c(q, _R1)[1] * D
            i2 = r2 * D
            lanes = lax.iota(jnp.int32, L) + j * L
            ob = j * L

            @plsc.parallel_loop(0, 8)
            def p12(dt):
                for fi in range(8):
                    d = dt * 8 + fi
                    v1 = plsc.load_gather(w1_v, [i1 + d])
                    out_v[dt, fi, pl.ds(ob, L)] = v1
                    v2 = plsc.load_gather(w2_v, [i2 + d])
                    out_v[8 + dt, fi, pl.ds(ob, L)] = v2

            @plsc.parallel_loop(0, D, unroll=8,
                                carry=jnp.zeros((L,), jnp.float32))
            def acc(d, a):
                v = plsc.load_gather(rows_v, [lanes, jnp.full((L,), d)])
                return a + v * v

            s3 = _scale_from_normsq(acc)

            @plsc.parallel_loop(0, 8)
            def p3(dt):
                for fi in range(8):
                    d = dt * 8 + fi
                    v3 = plsc.load_gather(rows_v, [lanes, _cc(d)])
                    out_v[16 + dt, fi, pl.ds(ob, L)] = v3 * s3

            return 0

        lax.fori_loop(0, GRP, j_body, 0)

    def pair(g, _):
        for b in range(2):
            s = g * 2 + b
            @pl.when(s + 1 < STEPS)
            def _():
                _gather(s + 1, 1 - b).start()
            _gather(s, b).wait()
            @pl.when(s >= 2)
            def _():
                pltpu.make_async_copy(out_b[b], _dst(s - 2), osem_b[b]).wait()
            compute(s, b)
            pltpu.make_async_copy(out_b[b], _dst(s), osem_b[b]).start()
        return 0

    lax.fori_loop(0, STEPS // 2, pair, 0)
    pltpu.make_async_copy(out_b[0], _dst(STEPS - 2), osem_b[0]).wait()
    pltpu.make_async_copy(out_b[1], _dst(STEPS - 1), osem_b[1]).wait()


_sc_embed = functools.partial(
    pl.kernel,
    out_type=jax.ShapeDtypeStruct((HIST, FT, NBT, 8, CHUNK), jnp.float32),
    mesh=plsc.VectorSubcoreMesh(core_axis_name="c", subcore_axis_name="s"),
    compiler_params=pltpu.CompilerParams(
        needs_layout_passes=False, use_tc_tiling_on_sc=False),
    scratch_types=[
        pltpu.VMEM((_R1 * D,), jnp.float32),
        pltpu.VMEM((_R2 * D,), jnp.float32),
        pltpu.VMEM((PER_W,), jnp.float32),
        pltpu.VMEM((STEPS, CHUNK), jnp.int32),
        pltpu.VMEM((CHUNK, D), jnp.float32),
        pltpu.VMEM((CHUNK, D), jnp.float32),
        pltpu.VMEM((FT, 8, CHUNK), jnp.float32),
        pltpu.VMEM((FT, 8, CHUNK), jnp.float32),
        pltpu.SemaphoreType.DMA,
        pltpu.SemaphoreType.DMA,
        pltpu.SemaphoreType.DMA,
        pltpu.SemaphoreType.DMA,
    ],
)(_body)


def kernel(idx, W1, W2, W3):
    idxt = idx.reshape(BATCH, HIST).T.reshape(N)
    out5 = _sc_embed(idxt, W1.reshape(_R1 * D), W2.reshape(_R2 * D), W3)
    return out5.transpose(2, 4, 0, 1, 3).reshape(BATCH, HIST, DOUT)
